# Initial kernel scaffold; baseline (speedup 1.0000x reference)
import jax
import jax.numpy as jnp
from jax.experimental import pallas as pl


def kernel(sparse, dense, tables, W1, b1, g1, be1, W2, b2, g2, be2, Wout, bout):
    raise NotImplementedError("write your pallas kernel here")


# speedup vs baseline: 1.2455x; 1.2455x over previous
import jax
import jax.numpy as jnp
import numpy as np
from jax.experimental import pallas as pl

N_FIELDS = 26
VOCAB = 100000
EMB = 16
DENSE = 13
B = 16384
H1 = 400
H2 = 400
BN_EPS = 1e-5
FE = N_FIELDS * EMB

_PERM = (np.arange(FE) % N_FIELDS) * EMB + np.arange(FE) // N_FIELDS


def _tc_body(embt_ref, denset_ref, w1t_ref, w1dt_ref, b1_ref, w2t_ref, b2_ref,
             sel_ref, ones_ref, wht_ref, c_ref, out_ref):
    et = embt_ref[...]
    CB = et.shape[1] // N_FIELDS
    ew = et.reshape(FE, CB)
    x1 = jnp.dot(w1t_ref[...], ew, preferred_element_type=jnp.float32)
    x1 += jnp.dot(w1dt_ref[...], denset_ref[...], preferred_element_type=jnp.float32)
    h1 = jnp.maximum(x1 + b1_ref[...], 0.0)
    x2 = jnp.dot(w2t_ref[...], h1, preferred_element_type=jnp.float32)
    h2 = jnp.maximum(x2 + b2_ref[...], 0.0)
    p = jnp.dot(sel_ref[...], ew, preferred_element_type=jnp.float32)
    ssum = p[:EMB, :]
    fm1 = p[EMB:, :]
    sumsq = jnp.dot(ones_ref[...], ew * ew, preferred_element_type=jnp.float32)
    fm2 = 0.5 * (jnp.sum(ssum * ssum, axis=0, keepdims=True) - sumsq)
    logit = (fm1
             + jnp.dot(wht_ref[...], h2, preferred_element_type=jnp.float32)
             + fm2 * c_ref[0, 0] + c_ref[0, 1])
    out_ref[...] = jax.nn.sigmoid(logit)


def _tc_mlp_t(embt, denset, w1t, w1dt, b1c, w2t, b2c, sel, ones, wht, consts):
    CB = denset.shape[1]
    full = lambda shape: pl.BlockSpec(shape, lambda: tuple(0 for _ in shape))
    return pl.pallas_call(
        _tc_body,
        grid=(),
        in_specs=[
            full((EMB, N_FIELDS * CB)),
            full((DENSE, CB)),
            full((H1, FE)),
            full((H1, DENSE)),
            full((H1, 1)),
            full((H2, H1)),
            full((H2, 1)),
            full((EMB + 1, FE)),
            full((1, FE)),
            full((1, H2)),
            full((1, 2)),
        ],
        out_specs=full((1, CB)),
        out_shape=jax.ShapeDtypeStruct((1, CB), jnp.float32),
    )(embt, denset, w1t, w1dt, b1c, w2t, b2c, sel, ones, wht, consts)


def kernel(sparse, dense, tables, W1, b1, g1, be1, W2, b2, g2, be2, Wout, bout):
    def _gather_t(idx_rows):
        n = idx_rows.shape[0]
        stack = jax.vmap(lambda t, idx: t.at[idx].get(mode='promise_in_bounds'),
                         in_axes=(0, 1), out_axes=0)(tables, idx_rows)
        return stack.reshape(N_FIELDS * n, EMB).T

    inv = 1.0 / jnp.sqrt(1.0 + BN_EPS)
    s1 = inv * g1
    b1c = ((b1 * s1 + be1))[:, None]
    s2 = inv * g2
    w2t = (W2 * s2[None, :]).T
    b2c = ((b2 * s2 + be2))[:, None]
    perm = jnp.asarray(_PERM)
    w1t = (W1[:FE, :] * s1[None, :])[perm, :].T
    w1dt = (W1[FE:, :] * s1[None, :]).T
    sel_np = np.zeros((EMB + 1, FE), dtype=np.float32)
    for e2 in range(EMB):
        sel_np[e2, e2 * N_FIELDS:(e2 + 1) * N_FIELDS] = 1.0
    sel = jnp.asarray(sel_np).at[EMB, :].set(
        jnp.tile(Wout[:N_FIELDS, 0], EMB))
    ones = jnp.ones((1, FE), dtype=jnp.float32)
    wht = Wout[N_FIELDS + 1:, :].T
    consts = jnp.stack([Wout[N_FIELDS, 0], bout[0]])[None, :]

    denset = dense.T

    SPLIT = 4
    CB = B // SPLIT
    outs = []
    for s in range(SPLIT):
        embt_s = _gather_t(jax.lax.slice_in_dim(sparse, s * CB, (s + 1) * CB, axis=0))
        denset_s = jax.lax.slice_in_dim(denset, s * CB, (s + 1) * CB, axis=1)
        outs.append(_tc_mlp_t(embt_s, denset_s, w1t, w1dt, b1c, w2t, b2c,
                              sel, ones, wht, consts))
    return jnp.concatenate(outs, axis=1).reshape(B)
